# baseline (speedup 1.0000x reference)
import functools

import jax
import jax.numpy as jnp
from jax import lax
from jax.experimental import pallas as pl
from jax.experimental.pallas import tpu as pltpu
from jax.experimental.pallas import tpu_sc as plsc

N = 10000
E = 320000
FB = 128
K = 128
NW = 32
CPT = 80
EPAD = NW * CPT * K
NITER = CPT // 4
ROWB = K * FB * 4

_mesh = plsc.VectorSubcoreMesh(
    core_axis_name="c", subcore_axis_name="s", num_cores=2, num_subcores=16)


@functools.partial(
    pl.kernel,
    out_type=jax.ShapeDtypeStruct((2, N, FB), jnp.float32),
    mesh=_mesh,
    scratch_types=[
        pltpu.VMEM((K, FB), jnp.float32),
        pltpu.VMEM((K, FB), jnp.float32),
        pltpu.VMEM((CPT, K), jnp.int32),
        pltpu.VMEM((K,), jnp.int32),
        pltpu.VMEM((K,), jnp.int32),
        pltpu.VMEM((K,), jnp.float32),
        pltpu.VMEM((K,), jnp.float32),
        pltpu.VMEM_SHARED((N, FB), jnp.float32),
        pltpu.SemaphoreType.DMA,
        pltpu.SemaphoreType.DMA,
        pltpu.SemaphoreType.DMA,
        pltpu.SemaphoreType.DMA,
        pltpu.SemaphoreType.DMA,
    ],
)
def _spmm_block(h_hbm, col_hbm, dst_hbm, vals_hbm, out_hbm,
                rows0, rows1, dsts_2d, col_v0, col_v1, vals_v0, vals_v1,
                acc, sem0, sem1, semi0, semi1, sem_d):
    cid = lax.axis_index("c")
    sid = lax.axis_index("s")
    wid = cid * 16 + sid
    bufs = (rows0, rows1)
    sems = (sem0, sem1)
    col_vs = (col_v0, col_v1)
    vals_vs = (vals_v0, vals_v1)
    semis = (semi0, semi1)

    d_dst = pltpu.async_copy(
        dst_hbm.at[pl.ds(pl.multiple_of(wid * CPT, 8), CPT)], dsts_2d, sem_d)

    def idxload(c, b):
        off = pl.multiple_of((wid * CPT + c) * K, 8)
        pltpu.async_copy(col_hbm.at[pl.ds(off, K)], col_vs[b], semis[b])
        pltpu.async_copy(vals_hbm.at[pl.ds(off, K)], vals_vs[b], semis[b])

    def wait_idx(b):
        pltpu.make_async_copy(col_hbm.at[pl.ds(0, K)], col_vs[b],
                              semis[b]).wait()
        pltpu.make_async_copy(vals_hbm.at[pl.ds(0, K)], vals_vs[b],
                              semis[b]).wait()

    idxload(0, 0)
    idxload(1, 1)

    def zrow(i, carry):
        for j in range(FB // 16):
            rows0[i, pl.ds(j * 16, 16)] = jnp.zeros((16,), jnp.float32)
        return carry
    lax.fori_loop(0, K, zrow, 0)
    zbase = sid * 624

    @pl.when(sid < 15)
    def _():
        def zc(k, carry):
            off = pl.multiple_of(zbase + k * 104, 8)
            pltpu.sync_copy(rows0.at[pl.ds(0, 104)], acc.at[pl.ds(off, 104)])
            return carry
        lax.fori_loop(0, 6, zc, 0)

    @pl.when(sid == 15)
    def _():
        def zc(k, carry):
            off = pl.multiple_of(zbase + k * 128, 8)
            pltpu.sync_copy(rows0, acc.at[pl.ds(off, 128)])
            return carry
        lax.fori_loop(0, 5, zc, 0)

    def gather(c, b):
        pltpu.async_copy(h_hbm.at[col_vs[b]], bufs[b], sems[b])

    def scatter(c, b):
        pltpu.async_copy(bufs[b], acc.at[dsts_2d.at[c]], sems[b], add=True)

    def wait_rows(b):
        pltpu.make_async_copy(h_hbm.at[col_vs[b]], bufs[b], sems[b]).wait()

    def scale(c, b):
        rv = bufs[b]
        vv = vals_vs[b]

        def sg(g, carry):
            vbase = pl.multiple_of(g * 16, 16)
            vals16 = vv[pl.ds(vbase, 16)]
            for l in range(16):
                e = g * 16 + l
                v = vals16.at[jnp.full((16,), l, jnp.int32)].get(
                    mode="promise_in_bounds")
                for j in range(FB // 16):
                    sl = pl.ds(j * 16, 16)
                    rv[e, sl] = rv[e, sl] * v
            return carry
        lax.fori_loop(0, K // 16, sg, 0)

    wait_idx(0)
    gather(0, 0)
    wait_idx(1)
    gather(1, 1)
    d_dst.wait()
    plsc.subcore_barrier()

    def body(k, last=False):
        c0 = 2 * k
        c1 = c0 + 1
        for b, c in ((0, c0), (1, c1)):
            wait_rows(b)
            scale(c, b)
            if not last:
                idxload(c + 2, b)
            scatter(c, b)
        for b, c in ((0, c0), (1, c1)):
            wait_rows(b)
            if not last:
                wait_idx(b)
                gather(c + 2, b)

    def loop_body(k, carry):
        body(k)
        return carry
    lax.fori_loop(0, CPT // 2 - 1, loop_body, 0)
    body(CPT // 2 - 1, last=True)
    plsc.subcore_barrier()

    off = pl.multiple_of(zbase, 8)

    @pl.when(sid < 15)
    def _():
        pltpu.sync_copy(acc.at[pl.ds(off, 624)],
                        out_hbm.at[cid, pl.ds(off, 624)])

    @pl.when(sid == 15)
    def _():
        pltpu.sync_copy(acc.at[pl.ds(off, 640)],
                        out_hbm.at[cid, pl.ds(off, 640)])


R = 400


def _tc1_body(p_ref, w1_ref, b1_ref, w2b_ref, out_ref, h1_s):
    b = pl.program_id(1)

    @pl.when(b == 0)
    def _():
        ax = p_ref[0] + p_ref[1]
        h1 = jnp.tanh(
            jnp.dot(ax, w1_ref[...], preferred_element_type=jnp.float32)
            + b1_ref[...])
        h1_s[...] = h1

    out_ref[0] = jnp.dot(h1_s[...], w2b_ref[0],
                         preferred_element_type=jnp.float32)


_tc1 = pl.pallas_call(
    _tc1_body,
    grid=(N // R, 4),
    in_specs=[
        pl.BlockSpec((2, R, 128), lambda i, b: (0, i, 0)),
        pl.BlockSpec((128, 1024), lambda i, b: (0, 0)),
        pl.BlockSpec((1, 1024), lambda i, b: (0, 0)),
        pl.BlockSpec((1, 1024, 128), lambda i, b: (b, 0, 0)),
    ],
    out_specs=pl.BlockSpec((1, R, 128), lambda i, b: (b, i, 0)),
    out_shape=jax.ShapeDtypeStruct((4, N, 128), jnp.float32),
    scratch_shapes=[pltpu.VMEM((R, 1024), jnp.float32)],
)


def _tc2_body(q0, q1, q2, q3, b2_ref, w3_ref, out_ref):
    qs = (q0, q1, q2, q3)
    parts = []
    for k in range(4):
        parts.append(jnp.tanh(qs[k][0] + qs[k][1]
                              + b2_ref[0, pl.ds(k * 128, 128)][None, :]))
    h2 = jnp.concatenate(parts, axis=1)
    out_ref[...] = jnp.dot(h2, w3_ref[...], preferred_element_type=jnp.float32)


_tc2 = pl.pallas_call(
    _tc2_body,
    grid=(N // R,),
    in_specs=[
        pl.BlockSpec((2, R, 128), lambda i: (0, i, 0)),
        pl.BlockSpec((2, R, 128), lambda i: (0, i, 0)),
        pl.BlockSpec((2, R, 128), lambda i: (0, i, 0)),
        pl.BlockSpec((2, R, 128), lambda i: (0, i, 0)),
        pl.BlockSpec((1, 512), lambda i: (0, 0)),
        pl.BlockSpec((512, 128), lambda i: (0, 0)),
    ],
    out_specs=pl.BlockSpec((R, 128), lambda i: (i, 0)),
    out_shape=jax.ShapeDtypeStruct((N, 128), jnp.float32),
)


def _tc3_body(r_ref, b3_ref, wf1t, bf1_ref, wf2t, bf2_ref, wf3t, out_ref):
    g = r_ref[0] + r_ref[1] + b3_ref[...]
    d = jnp.maximum(
        jnp.dot(g, wf1t[...], preferred_element_type=jnp.float32)
        + bf1_ref[...], 0.0)
    d = jnp.maximum(
        jnp.dot(d, wf2t[...], preferred_element_type=jnp.float32)
        + bf2_ref[...], 0.0)
    out_ref[...] = jnp.dot(d, wf3t[...], preferred_element_type=jnp.float32)


_tc3 = pl.pallas_call(
    _tc3_body,
    grid=(N // R,),
    in_specs=[
        pl.BlockSpec((2, R, 128), lambda i: (0, i, 0)),
        pl.BlockSpec((1, 128), lambda i: (0, 0)),
        pl.BlockSpec((128, 152), lambda i: (0, 0)),
        pl.BlockSpec((1, 152), lambda i: (0, 0)),
        pl.BlockSpec((152, 48), lambda i: (0, 0)),
        pl.BlockSpec((1, 48), lambda i: (0, 0)),
        pl.BlockSpec((48, 128), lambda i: (0, 0)),
    ],
    out_specs=pl.BlockSpec((R, 128), lambda i: (i, 0)),
    out_shape=jax.ShapeDtypeStruct((N, 128), jnp.float32),
)


def kernel(x, edge_index, adj_vals, W1, b1, W2, b2, W3, b3,
           Wf1, bf1, Wf2, bf2, Wf3, bf3):
    npad = EPAD - E
    dst = jnp.concatenate(
        [edge_index[0].astype(jnp.int32), jnp.zeros((npad,), jnp.int32)]
    ).reshape(NW * CPT, K)
    col = jnp.concatenate(
        [edge_index[1].astype(jnp.int32), jnp.zeros((npad,), jnp.int32)])
    vals = jnp.concatenate(
        [adj_vals.astype(jnp.float32), jnp.zeros((npad,), jnp.float32)])

    p1 = _spmm_block(x, col, dst, vals)
    w2b = W2.reshape(1024, 4, 128).transpose(1, 0, 2)
    t2 = _tc1(p1, W1, b1.reshape(1, 1024), w2b)

    q = [_spmm_block(t2[k], col, dst, vals) for k in range(4)]
    t3 = _tc2(q[0], q[1], q[2], q[3], b2.reshape(1, 512), W3)

    r = _spmm_block(t3, col, dst, vals)
    wf3t = jnp.zeros((48, 128), jnp.float32).at[:, :1].set(Wf3.T)
    out128 = _tc3(r, b3.reshape(1, 128), Wf1.T, bf1.reshape(1, 152),
                  Wf2.T, bf2.reshape(1, 48), wf3t)
    return out128[:, :1] + bf3[0]


# speedup vs baseline: 2.6762x; 2.6762x over previous
import functools

import jax
import jax.numpy as jnp
from jax import lax
from jax.experimental import pallas as pl
from jax.experimental.pallas import tpu as pltpu
from jax.experimental.pallas import tpu_sc as plsc

N = 10000
E = 320000
FB = 128
K = 128
NW = 32
CPT = 80
EPAD = NW * CPT * K
NITER = CPT // 4
ROWB = K * FB * 4

_mesh = plsc.VectorSubcoreMesh(
    core_axis_name="c", subcore_axis_name="s", num_cores=2, num_subcores=16)


@functools.partial(
    pl.kernel,
    out_type=jax.ShapeDtypeStruct((2, N, FB), jnp.float32),
    mesh=_mesh,
    scratch_types=[
        pltpu.VMEM((K, FB), jnp.float32),
        pltpu.VMEM((K, FB), jnp.float32),
        pltpu.VMEM((CPT, K), jnp.int32),
        pltpu.VMEM((K,), jnp.int32),
        pltpu.VMEM((K,), jnp.int32),
        pltpu.VMEM((K,), jnp.float32),
        pltpu.VMEM((K,), jnp.float32),
        pltpu.VMEM_SHARED((N, FB), jnp.float32),
        pltpu.SemaphoreType.DMA,
        pltpu.SemaphoreType.DMA,
        pltpu.SemaphoreType.DMA,
        pltpu.SemaphoreType.DMA,
        pltpu.SemaphoreType.DMA,
    ],
)
def _spmm_block(h_hbm, col_hbm, dst_hbm, vals_hbm, out_hbm,
                rows0, rows1, dsts_2d, col_v0, col_v1, vals_v0, vals_v1,
                acc, sem0, sem1, semi0, semi1, sem_d):
    cid = lax.axis_index("c")
    sid = lax.axis_index("s")
    wid = cid * 16 + sid
    bufs = (rows0, rows1)
    sems = (sem0, sem1)
    col_vs = (col_v0, col_v1)
    vals_vs = (vals_v0, vals_v1)
    semis = (semi0, semi1)

    d_dst = pltpu.async_copy(
        dst_hbm.at[pl.ds(pl.multiple_of(wid * CPT, 8), CPT)], dsts_2d, sem_d)

    def idxload(c, b):
        off = pl.multiple_of((wid * CPT + c) * K, 8)
        pltpu.async_copy(col_hbm.at[pl.ds(off, K)], col_vs[b], semis[b])
        pltpu.async_copy(vals_hbm.at[pl.ds(off, K)], vals_vs[b], semis[b])

    def wait_idx(b):
        pltpu.make_async_copy(col_hbm.at[pl.ds(0, K)], col_vs[b],
                              semis[b]).wait()
        pltpu.make_async_copy(vals_hbm.at[pl.ds(0, K)], vals_vs[b],
                              semis[b]).wait()

    idxload(0, 0)
    idxload(1, 1)

    def zrow(i, carry):
        for j in range(FB // 16):
            rows0[i, pl.ds(j * 16, 16)] = jnp.zeros((16,), jnp.float32)
        return carry
    lax.fori_loop(0, K, zrow, 0)
    zbase = sid * 624

    @pl.when(sid < 15)
    def _():
        def zc(k, carry):
            off = pl.multiple_of(zbase + k * 104, 8)
            pltpu.sync_copy(rows0.at[pl.ds(0, 104)], acc.at[pl.ds(off, 104)])
            return carry
        lax.fori_loop(0, 6, zc, 0)

    @pl.when(sid == 15)
    def _():
        def zc(k, carry):
            off = pl.multiple_of(zbase + k * 128, 8)
            pltpu.sync_copy(rows0, acc.at[pl.ds(off, 128)])
            return carry
        lax.fori_loop(0, 5, zc, 0)

    def gather(c, b):
        pltpu.async_copy(h_hbm.at[col_vs[b]], bufs[b], sems[b])

    def scatter(c, b):
        pltpu.async_copy(bufs[b], acc.at[dsts_2d.at[c]], sems[b], add=True)

    def wait_rows(b):
        pltpu.make_async_copy(h_hbm.at[col_vs[b]], bufs[b], sems[b]).wait()

    def scale(c, b):
        rv = bufs[b]
        vv = vals_vs[b]

        def sg(g, carry):
            vbase = pl.multiple_of(g * 16, 16)
            vals16 = vv[pl.ds(vbase, 16)]
            for l in range(16):
                e = g * 16 + l
                v = vals16.at[jnp.full((16,), l, jnp.int32)].get(
                    mode="promise_in_bounds")
                for j in range(FB // 16):
                    sl = pl.ds(j * 16, 16)
                    rv[e, sl] = rv[e, sl] * v
            return carry
        lax.fori_loop(0, K // 16, sg, 0)

    wait_idx(0)
    gather(0, 0)
    wait_idx(1)
    gather(1, 1)
    d_dst.wait()
    plsc.subcore_barrier()

    def body(k, last=False):
        c0 = 2 * k
        c1 = c0 + 1
        for b, c in ((0, c0), (1, c1)):
            wait_rows(b)
            scale(c, b)
            if not last:
                idxload(c + 2, b)
            scatter(c, b)
        for b, c in ((0, c0), (1, c1)):
            wait_rows(b)
            if not last:
                wait_idx(b)
                gather(c + 2, b)

    def loop_body(k, carry):
        body(k)
        return carry
    lax.fori_loop(0, CPT // 2 - 1, loop_body, 0)
    body(CPT // 2 - 1, last=True)
    plsc.subcore_barrier()

    off = pl.multiple_of(zbase, 8)

    @pl.when(sid < 15)
    def _():
        pltpu.sync_copy(acc.at[pl.ds(off, 624)],
                        out_hbm.at[cid, pl.ds(off, 624)])

    @pl.when(sid == 15)
    def _():
        pltpu.sync_copy(acc.at[pl.ds(off, 640)],
                        out_hbm.at[cid, pl.ds(off, 640)])


R = 400


def _tc1_body(p_ref, w1_ref, b1_ref, w2b_ref, out_ref, h1_s):
    b = pl.program_id(1)

    @pl.when(b == 0)
    def _():
        ax = p_ref[0] + p_ref[1]
        h1 = jnp.tanh(
            jnp.dot(ax, w1_ref[...], preferred_element_type=jnp.float32)
            + b1_ref[...])
        h1_s[...] = h1

    out_ref[0] = jnp.dot(h1_s[...], w2b_ref[0],
                         preferred_element_type=jnp.float32)


_tc1 = pl.pallas_call(
    _tc1_body,
    grid=(N // R, 4),
    in_specs=[
        pl.BlockSpec((2, R, 128), lambda i, b: (0, i, 0)),
        pl.BlockSpec((128, 1024), lambda i, b: (0, 0)),
        pl.BlockSpec((1, 1024), lambda i, b: (0, 0)),
        pl.BlockSpec((1, 1024, 128), lambda i, b: (b, 0, 0)),
    ],
    out_specs=pl.BlockSpec((1, R, 128), lambda i, b: (b, i, 0)),
    out_shape=jax.ShapeDtypeStruct((4, N, 128), jnp.float32),
    scratch_shapes=[pltpu.VMEM((R, 1024), jnp.float32)],
)


def _tc2_body(q0, q1, q2, q3, b2_ref, w3_ref, out_ref):
    qs = (q0, q1, q2, q3)
    parts = []
    for k in range(4):
        parts.append(jnp.tanh(qs[k][0] + qs[k][1]
                              + b2_ref[0, pl.ds(k * 128, 128)][None, :]))
    h2 = jnp.concatenate(parts, axis=1)
    out_ref[...] = jnp.dot(h2, w3_ref[...], preferred_element_type=jnp.float32)


_tc2 = pl.pallas_call(
    _tc2_body,
    grid=(N // R,),
    in_specs=[
        pl.BlockSpec((2, R, 128), lambda i: (0, i, 0)),
        pl.BlockSpec((2, R, 128), lambda i: (0, i, 0)),
        pl.BlockSpec((2, R, 128), lambda i: (0, i, 0)),
        pl.BlockSpec((2, R, 128), lambda i: (0, i, 0)),
        pl.BlockSpec((1, 512), lambda i: (0, 0)),
        pl.BlockSpec((512, 128), lambda i: (0, 0)),
    ],
    out_specs=pl.BlockSpec((R, 128), lambda i: (i, 0)),
    out_shape=jax.ShapeDtypeStruct((N, 128), jnp.float32),
)


def _tc3_body(r_ref, b3_ref, wf1t, bf1_ref, wf2t, bf2_ref, wf3t, out_ref):
    g = r_ref[0] + r_ref[1] + b3_ref[...]
    d = jnp.maximum(
        jnp.dot(g, wf1t[...], preferred_element_type=jnp.float32)
        + bf1_ref[...], 0.0)
    d = jnp.maximum(
        jnp.dot(d, wf2t[...], preferred_element_type=jnp.float32)
        + bf2_ref[...], 0.0)
    out_ref[...] = jnp.dot(d, wf3t[...], preferred_element_type=jnp.float32)


_tc3 = pl.pallas_call(
    _tc3_body,
    grid=(N // R,),
    in_specs=[
        pl.BlockSpec((2, R, 128), lambda i: (0, i, 0)),
        pl.BlockSpec((1, 128), lambda i: (0, 0)),
        pl.BlockSpec((128, 152), lambda i: (0, 0)),
        pl.BlockSpec((1, 152), lambda i: (0, 0)),
        pl.BlockSpec((152, 48), lambda i: (0, 0)),
        pl.BlockSpec((1, 48), lambda i: (0, 0)),
        pl.BlockSpec((48, 128), lambda i: (0, 0)),
    ],
    out_specs=pl.BlockSpec((R, 128), lambda i: (i, 0)),
    out_shape=jax.ShapeDtypeStruct((N, 128), jnp.float32),
)


def kernel(x, edge_index, adj_vals, W1, b1, W2, b2, W3, b3,
           Wf1, bf1, Wf2, bf2, Wf3, bf3):
    npad = EPAD - E
    spread = jnp.arange(npad, dtype=jnp.int32) % N
    dst = jnp.concatenate(
        [edge_index[0].astype(jnp.int32), spread]).reshape(NW * CPT, K)
    col = jnp.concatenate([edge_index[1].astype(jnp.int32), spread])
    vals = jnp.concatenate(
        [adj_vals.astype(jnp.float32), jnp.zeros((npad,), jnp.float32)])

    p1 = _spmm_block(x, col, dst, vals)
    w2b = W2.reshape(1024, 4, 128).transpose(1, 0, 2)
    t2 = _tc1(p1, W1, b1.reshape(1, 1024), w2b)

    q = [_spmm_block(t2[k], col, dst, vals) for k in range(4)]
    t3 = _tc2(q[0], q[1], q[2], q[3], b2.reshape(1, 512), W3)

    r = _spmm_block(t3, col, dst, vals)
    wf3t = jnp.zeros((48, 128), jnp.float32).at[:, :1].set(Wf3.T)
    out128 = _tc3(r, b3.reshape(1, 128), Wf1.T, bf1.reshape(1, 152),
                  Wf2.T, bf2.reshape(1, 48), wf3t)
    return out128[:, :1] + bf3[0]
